# baseline (speedup 1.0000x reference)
import functools

import jax
import jax.numpy as jnp
from jax import lax
from jax.experimental import pallas as pl
from jax.experimental.pallas import tpu as pltpu
from jax.experimental.pallas import tpu_sc as plsc

B, S, D = 4, 4096, 1024
N = B * S
NC, NS = 2, 16
NW = NC * NS
TPW = N // NW
C = 32
NCHUNK = TPW // C
LANES = 16


def _body(tokens_hbm, idx_hbm, table_hbm, out_hbm, idx_v, tok_v, emb_v, sem):
    wid = lax.axis_index("s") * NC + lax.axis_index("c")
    base = wid * TPW
    pltpu.sync_copy(idx_hbm.at[pl.ds(base, TPW)], idx_v)

    def chunk_body(c, carry):
        start = base + c * C
        pltpu.sync_copy(tokens_hbm.at[pl.ds(start, C)], tok_v)
        pltpu.async_copy(table_hbm.at[idx_v.at[pl.ds(c * C, C)]], emb_v, sem).wait()

        def row_body(i, carry2):
            for j in range(D // LANES):
                sl = pl.ds(j * LANES, LANES)
                tok_v[i, sl] = tok_v[i, sl] + emb_v[i, sl]
            return carry2

        lax.fori_loop(0, C, row_body, 0)
        pltpu.sync_copy(tok_v, out_hbm.at[pl.ds(start, C)])
        return carry

    lax.fori_loop(0, NCHUNK, chunk_body, 0)


@functools.partial(jax.jit, static_argnames=())
def _run(tokens2d, idx1d, table):
    mesh = plsc.VectorSubcoreMesh(core_axis_name="c", subcore_axis_name="s")
    f = pl.kernel(
        _body,
        out_type=jax.ShapeDtypeStruct((N, D), jnp.float32),
        mesh=mesh,
        scratch_types=[
            pltpu.VMEM((TPW,), jnp.int32),
            pltpu.VMEM((C, D), jnp.float32),
            pltpu.VMEM((C, D), jnp.float32),
            pltpu.SemaphoreType.DMA,
        ],
    )
    return f(tokens2d, idx1d, table)


def kernel(tokens, example_positions, pos_table):
    tokens2d = tokens.reshape(N, D)
    idx1d = example_positions.reshape(N).astype(jnp.int32)
    out = _run(tokens2d, idx1d, pos_table)
    return out.reshape(B, S, D)
